# Initial kernel scaffold; baseline (speedup 1.0000x reference)
import jax
import jax.numpy as jnp
from jax import lax
from jax.experimental import pallas as pl
from jax.experimental.pallas import tpu as pltpu
from jax.experimental.pallas import tpu_sc as plsc

NC = 2
NS = 16
CH = 128
LW = 16


def _deg_body(dst_hbm, zeros_hbm, ones_hbm, out_hbm, idx_v, ones_v, acc):
    c = lax.axis_index("c")
    s = lax.axis_index("s")
    n1 = acc.shape[0]
    spt = n1 // NS
    nchunk = idx_v.shape[0]
    pltpu.sync_copy(zeros_hbm.at[pl.ds(s * spt, spt)], acc.at[pl.ds(s * spt, spt)])
    pltpu.sync_copy(ones_hbm, ones_v)
    pltpu.sync_copy(dst_hbm.at[c, s], idx_v)
    plsc.subcore_barrier()

    def step(j, carry):
        pltpu.sync_copy(ones_v, acc.at[idx_v.at[j]], add=True)
        return carry

    lax.fori_loop(0, nchunk, step, 0)
    plsc.subcore_barrier()
    pltpu.sync_copy(acc.at[pl.ds(s * spt, spt)], out_hbm.at[c, pl.ds(s * spt, spt)])


def _prop_body(table_hbm, srcoff_hbm, dst_hbm, out_hbm, idxs_v, idxd_v, rows_v, acc, sem):
    c = lax.axis_index("c")
    s = lax.axis_index("s")
    n1 = acc.shape[0]
    spt = n1 // NS
    nchunk = idxs_v.shape[0]
    pltpu.sync_copy(table_hbm.at[pl.ds(c * n1 + s * spt, spt)],
                    acc.at[pl.ds(s * spt, spt)])
    pltpu.sync_copy(srcoff_hbm.at[c, s], idxs_v)
    pltpu.sync_copy(dst_hbm.at[s], idxd_v)
    plsc.subcore_barrier()

    def step(j, carry):
        pltpu.async_copy(table_hbm.at[idxs_v.at[j]], rows_v, sem).wait()
        pltpu.sync_copy(rows_v, acc.at[idxd_v.at[j]], add=True)
        return carry

    lax.fori_loop(0, nchunk, step, 0)
    plsc.subcore_barrier()
    pltpu.sync_copy(acc.at[pl.ds(s * spt, spt)], out_hbm.at[c, pl.ds(s * spt, spt)])


def _run_deg(dst_deg, zeros_n1, ones_ch, n1):
    mesh = plsc.VectorSubcoreMesh(core_axis_name="c", subcore_axis_name="s")
    dchunk = dst_deg.shape[2]
    return pl.kernel(
        _deg_body,
        out_type=jax.ShapeDtypeStruct((NC, n1, LW), jnp.float32),
        mesh=mesh,
        scratch_types=[
            pltpu.VMEM((dchunk, CH), jnp.int32),
            pltpu.VMEM((CH, LW), jnp.float32),
            pltpu.VMEM_SHARED((n1, LW), jnp.float32),
        ],
    )(dst_deg, zeros_n1, ones_ch)


def _run_prop(table, srcoff, dst_prop, n1, h):
    mesh = plsc.VectorSubcoreMesh(core_axis_name="c", subcore_axis_name="s")
    pchunk = srcoff.shape[2]
    return pl.kernel(
        _prop_body,
        out_type=jax.ShapeDtypeStruct((NC, n1, h), jnp.float32),
        mesh=mesh,
        scratch_types=[
            pltpu.VMEM((pchunk, CH), jnp.int32),
            pltpu.VMEM((pchunk, CH), jnp.int32),
            pltpu.VMEM((CH, h), jnp.float32),
            pltpu.VMEM_SHARED((n1, h), jnp.float32),
            pltpu.SemaphoreType.DMA,
        ],
    )(table, srcoff, dst_prop)


_TCB = 1024


def _tc1_body(x_ref, w_ref, degp_ref, tab_ref, dinv_ref):
    deg = 1.0 + degp_ref[0] + degp_ref[1]
    dinv = lax.rsqrt(deg)[:, 0:1]
    hmat = jnp.dot(x_ref[0], w_ref[0], preferred_element_type=jnp.float32)
    tab_ref[0] = hmat * dinv
    dinv_ref[...] = dinv


def _run_tc1(xs, ws, degp):
    _, n1, d = xs.shape
    h = ws.shape[2]
    nb = n1 // _TCB
    return pl.pallas_call(
        _tc1_body,
        grid=(2, nb),
        in_specs=[
            pl.BlockSpec((1, _TCB, d), lambda c, j: (c, j, 0)),
            pl.BlockSpec((1, d, h), lambda c, j: (c, 0, 0)),
            pl.BlockSpec((2, _TCB, LW), lambda c, j: (0, j, 0)),
        ],
        out_specs=[
            pl.BlockSpec((1, _TCB, h), lambda c, j: (c, j, 0)),
            pl.BlockSpec((_TCB, 1), lambda c, j: (j, 0)),
        ],
        out_shape=[
            jax.ShapeDtypeStruct((2, n1, h), jnp.float32),
            jax.ShapeDtypeStruct((n1, 1), jnp.float32),
        ],
    )(xs, ws, degp)


def _tc2_body(p_ref, dinv_ref, b_ref, a_ref, w_ref, tab_ref):
    dinv = dinv_ref[...]
    z = p_ref[0] * dinv + b_ref[...]
    g = jnp.where(z >= 0, z, a_ref[...] * z)
    tab_ref[0] = jnp.dot(g, w_ref[0], preferred_element_type=jnp.float32) * dinv


def _run_tc2(p, dinv, bs, as_, ws2):
    _, n1, h = p.shape
    nb = n1 // _TCB
    return pl.pallas_call(
        _tc2_body,
        grid=(2, nb),
        in_specs=[
            pl.BlockSpec((1, _TCB, h), lambda c, j: (c, j, 0)),
            pl.BlockSpec((_TCB, 1), lambda c, j: (j, 0)),
            pl.BlockSpec((1, h), lambda c, j: (c, 0)),
            pl.BlockSpec((1, h), lambda c, j: (c, 0)),
            pl.BlockSpec((1, h, h), lambda c, j: (c, 0, 0)),
        ],
        out_specs=pl.BlockSpec((1, _TCB, h), lambda c, j: (c, j, 0)),
        out_shape=jax.ShapeDtypeStruct((2, n1, h), jnp.float32),
    )(p, dinv, bs, as_, ws2)


def _tc3_body(p_ref, dinv_ref, b_ref, a_ref, w1_ref, w2_ref, x_ref):
    dinv = dinv_ref[...]

    def head(pc, bc, ac):
        z = pc * dinv + bc
        g = jnp.where(z >= 0, z, ac * z)
        nrm = jnp.sqrt(jnp.sum(g * g, axis=1, keepdims=True))
        return g / jnp.maximum(nrm, 1e-12)

    h1 = head(p_ref[0], b_ref[0:1], a_ref[0:1])
    h2 = head(p_ref[1], b_ref[1:2], a_ref[1:2])
    x_ref[...] = h1 * w1_ref[...] + h2 * w2_ref[...]


def _run_tc3(p, dinv, bs2, as2, w1p, w2p):
    _, n1, h = p.shape
    nb = n1 // _TCB
    return pl.pallas_call(
        _tc3_body,
        grid=(nb,),
        in_specs=[
            pl.BlockSpec((2, _TCB, h), lambda j: (0, j, 0)),
            pl.BlockSpec((_TCB, 1), lambda j: (j, 0)),
            pl.BlockSpec((2, h), lambda j: (0, 0)),
            pl.BlockSpec((2, h), lambda j: (0, 0)),
            pl.BlockSpec((_TCB, 1), lambda j: (j, 0)),
            pl.BlockSpec((_TCB, 1), lambda j: (j, 0)),
        ],
        out_specs=pl.BlockSpec((_TCB, h), lambda j: (j, 0)),
        out_shape=jax.ShapeDtypeStruct((n1, h), jnp.float32),
    )(p, dinv, bs2, as2, w1p, w2p)


def kernel(x1, x2, edge_index, adata, w, w1, w2,
           W1, b1, W2, b2, W3, b3, W4, b4, a1, a2, a3, a4):
    n, d = x1.shape
    h = W1.shape[1]
    e = edge_index.shape[1]

    n1 = ((n + LW + _TCB - 1) // _TCB) * _TCB

    grp = NC * NS * CH
    e_pad = ((e + grp - 1) // grp) * grp
    src = edge_index[0].astype(jnp.int32)
    dst = edge_index[1].astype(jnp.int32)
    fill = jnp.full((e_pad - e,), n, dtype=jnp.int32)
    src_p = jnp.concatenate([src, fill])
    dst_p = jnp.concatenate([dst, fill])

    srcoff = jnp.stack([src_p, src_p + n1]).reshape(NC, NS, e_pad // (NS * CH), CH)
    dst_prop = dst_p.reshape(NS, e_pad // (NS * CH), CH)
    dst_deg = dst_p.reshape(NC, NS, e_pad // grp, CH)

    zeros_n1 = jnp.zeros((n1, LW), jnp.float32)
    ones_ch = jnp.ones((CH, LW), jnp.float32)

    pad_rows = ((0, n1 - n), (0, 0))
    xs = jnp.stack([jnp.pad(x1, pad_rows), jnp.pad(x2, pad_rows)])
    ws_l1 = jnp.stack([W1, W2])
    ws_l2 = jnp.stack([W3, W4])
    bs_l1 = jnp.stack([b1, b2])
    bs_l2 = jnp.stack([b3, b4])
    as_l1 = jnp.stack([a1, a3])
    as_l2 = jnp.stack([a2, a4])
    w1p = jnp.pad(w1, pad_rows)
    w2p = jnp.pad(w2, pad_rows)

    degp = _run_deg(dst_deg, zeros_n1, ones_ch, n1)
    tab1, dinv = _run_tc1(xs, ws_l1, degp)
    p1 = _run_prop(tab1.reshape(NC * n1, h), srcoff, dst_prop, n1, h)
    tab2 = _run_tc2(p1, dinv, bs_l1, as_l1, ws_l2)
    p2 = _run_prop(tab2.reshape(NC * n1, h), srcoff, dst_prop, n1, h)
    x = _run_tc3(p2, dinv, bs_l2, as_l2, w1p, w2p)

    return (x[:n], w1, w2)


# speedup vs baseline: 8.7553x; 8.7553x over previous
import jax
import jax.numpy as jnp
from jax import lax
from jax.experimental import pallas as pl
from jax.experimental.pallas import tpu as pltpu
from jax.experimental.pallas import tpu_sc as plsc

NC = 2
NS = 16
CH = 128
GK = 8
LW = 128


def _deg_body(dst_hbm, zeros_hbm, ones_hbm, out_hbm, idx_v, ones_v, acc):
    c = lax.axis_index("c")
    s = lax.axis_index("s")
    n1 = acc.shape[0]
    spt = n1 // NS
    nchunk = idx_v.shape[0]
    pltpu.sync_copy(zeros_hbm.at[pl.ds(s * spt, spt)], acc.at[pl.ds(s * spt, spt)])
    pltpu.sync_copy(ones_hbm, ones_v)
    pltpu.sync_copy(dst_hbm.at[c, s], idx_v)
    plsc.subcore_barrier()

    def step(j, carry):
        pltpu.sync_copy(ones_v, acc.at[idx_v.at[j]], add=True)
        return carry

    lax.fori_loop(0, nchunk, step, 0)
    plsc.subcore_barrier()
    pltpu.sync_copy(acc.at[pl.ds(s * spt, spt)], out_hbm.at[c, pl.ds(s * spt, spt)])


def _prop_body(table_hbm, srcoff_hbm, dst_hbm, out_hbm, idxs_v, idxd_v, rows_v, acc, sem):
    c = lax.axis_index("c")
    s = lax.axis_index("s")
    n1 = acc.shape[0]
    spt = n1 // NS
    ngroup = srcoff_hbm.shape[2] // GK
    pltpu.sync_copy(table_hbm.at[pl.ds(c * n1 + s * spt, spt)],
                    acc.at[pl.ds(s * spt, spt)])
    plsc.subcore_barrier()

    def group(g, carry):
        pltpu.sync_copy(srcoff_hbm.at[c, s, pl.ds(g * GK, GK)], idxs_v)
        pltpu.sync_copy(dst_hbm.at[s, pl.ds(g * GK, GK)], idxd_v)
        for j in range(GK):
            pltpu.async_copy(table_hbm.at[idxs_v.at[j]], rows_v, sem).wait()
            pltpu.sync_copy(rows_v, acc.at[idxd_v.at[j]], add=True)
        return carry

    lax.fori_loop(0, ngroup, group, 0)
    plsc.subcore_barrier()
    pltpu.sync_copy(acc.at[pl.ds(s * spt, spt)], out_hbm.at[c, pl.ds(s * spt, spt)])


def _run_deg(dst_deg, zeros_n1, ones_ch, n1):
    mesh = plsc.VectorSubcoreMesh(core_axis_name="c", subcore_axis_name="s")
    dchunk = dst_deg.shape[2]
    return pl.kernel(
        _deg_body,
        out_type=jax.ShapeDtypeStruct((NC, n1, LW), jnp.float32),
        mesh=mesh,
        scratch_types=[
            pltpu.VMEM((dchunk, CH), jnp.int32),
            pltpu.VMEM((CH, LW), jnp.float32),
            pltpu.VMEM_SHARED((n1, LW), jnp.float32),
        ],
    )(dst_deg, zeros_n1, ones_ch)


def _run_prop(table, srcoff, dst_prop, n1, h):
    mesh = plsc.VectorSubcoreMesh(core_axis_name="c", subcore_axis_name="s")
    return pl.kernel(
        _prop_body,
        out_type=jax.ShapeDtypeStruct((NC, n1, h), jnp.float32),
        mesh=mesh,
        scratch_types=[
            pltpu.VMEM((GK, CH), jnp.int32),
            pltpu.VMEM((GK, CH), jnp.int32),
            pltpu.VMEM((CH, h), jnp.float32),
            pltpu.VMEM_SHARED((n1, h), jnp.float32),
            pltpu.SemaphoreType.DMA,
        ],
    )(table, srcoff, dst_prop)


_TCB = 1024


def _tc1_body(x_ref, w_ref, degp_ref, tab_ref, dinv_ref):
    deg = 1.0 + degp_ref[0] + degp_ref[1]
    dinv = lax.rsqrt(deg)[:, 0:1]
    hmat = jnp.dot(x_ref[0], w_ref[0], preferred_element_type=jnp.float32)
    tab_ref[0] = hmat * dinv
    dinv_ref[...] = dinv


def _run_tc1(xs, ws, degp):
    _, n1, d = xs.shape
    h = ws.shape[2]
    nb = n1 // _TCB
    return pl.pallas_call(
        _tc1_body,
        grid=(2, nb),
        in_specs=[
            pl.BlockSpec((1, _TCB, d), lambda c, j: (c, j, 0)),
            pl.BlockSpec((1, d, h), lambda c, j: (c, 0, 0)),
            pl.BlockSpec((2, _TCB, LW), lambda c, j: (0, j, 0)),
        ],
        out_specs=[
            pl.BlockSpec((1, _TCB, h), lambda c, j: (c, j, 0)),
            pl.BlockSpec((_TCB, 1), lambda c, j: (j, 0)),
        ],
        out_shape=[
            jax.ShapeDtypeStruct((2, n1, h), jnp.float32),
            jax.ShapeDtypeStruct((n1, 1), jnp.float32),
        ],
    )(xs, ws, degp)


def _tc2_body(p_ref, dinv_ref, b_ref, a_ref, w_ref, tab_ref):
    c = pl.program_id(0)
    dinv = dinv_ref[...]
    bvec = jnp.where(c == 0, b_ref[0:1], b_ref[1:2])
    avec = jnp.where(c == 0, a_ref[0:1], a_ref[1:2])
    z = p_ref[0] * dinv + bvec
    g = jnp.where(z >= 0, z, avec * z)
    tab_ref[0] = jnp.dot(g, w_ref[0], preferred_element_type=jnp.float32) * dinv


def _run_tc2(p, dinv, bs, as_, ws2):
    _, n1, h = p.shape
    nb = n1 // _TCB
    return pl.pallas_call(
        _tc2_body,
        grid=(2, nb),
        in_specs=[
            pl.BlockSpec((1, _TCB, h), lambda c, j: (c, j, 0)),
            pl.BlockSpec((_TCB, 1), lambda c, j: (j, 0)),
            pl.BlockSpec((2, h), lambda c, j: (0, 0)),
            pl.BlockSpec((2, h), lambda c, j: (0, 0)),
            pl.BlockSpec((1, h, h), lambda c, j: (c, 0, 0)),
        ],
        out_specs=pl.BlockSpec((1, _TCB, h), lambda c, j: (c, j, 0)),
        out_shape=jax.ShapeDtypeStruct((2, n1, h), jnp.float32),
    )(p, dinv, bs, as_, ws2)


def _tc3_body(p_ref, dinv_ref, b_ref, a_ref, w1_ref, w2_ref, x_ref):
    dinv = dinv_ref[...]

    def head(pc, bc, ac):
        z = pc * dinv + bc
        g = jnp.where(z >= 0, z, ac * z)
        nrm = jnp.sqrt(jnp.sum(g * g, axis=1, keepdims=True))
        return g / jnp.maximum(nrm, 1e-12)

    h1 = head(p_ref[0], b_ref[0:1], a_ref[0:1])
    h2 = head(p_ref[1], b_ref[1:2], a_ref[1:2])
    x_ref[...] = h1 * w1_ref[...] + h2 * w2_ref[...]


def _run_tc3(p, dinv, bs2, as2, w1p, w2p):
    _, n1, h = p.shape
    nb = n1 // _TCB
    return pl.pallas_call(
        _tc3_body,
        grid=(nb,),
        in_specs=[
            pl.BlockSpec((2, _TCB, h), lambda j: (0, j, 0)),
            pl.BlockSpec((_TCB, 1), lambda j: (j, 0)),
            pl.BlockSpec((2, h), lambda j: (0, 0)),
            pl.BlockSpec((2, h), lambda j: (0, 0)),
            pl.BlockSpec((_TCB, 1), lambda j: (j, 0)),
            pl.BlockSpec((_TCB, 1), lambda j: (j, 0)),
        ],
        out_specs=pl.BlockSpec((_TCB, h), lambda j: (j, 0)),
        out_shape=jax.ShapeDtypeStruct((n1, h), jnp.float32),
    )(p, dinv, bs2, as2, w1p, w2p)


def kernel(x1, x2, edge_index, adata, w, w1, w2,
           W1, b1, W2, b2, W3, b3, W4, b4, a1, a2, a3, a4):
    n, d = x1.shape
    h = W1.shape[1]
    e = edge_index.shape[1]

    n1 = ((n + LW + _TCB - 1) // _TCB) * _TCB

    grp = NS * CH * GK
    e_pad = ((e + grp - 1) // grp) * grp
    src = edge_index[0].astype(jnp.int32)
    dst = edge_index[1].astype(jnp.int32)
    fill = jnp.full((e_pad - e,), n, dtype=jnp.int32)
    src_p = jnp.concatenate([src, fill])
    dst_p = jnp.concatenate([dst, fill])

    srcoff = jnp.stack([src_p, src_p + n1]).reshape(NC, NS, e_pad // (NS * CH), CH)
    dst_prop = dst_p.reshape(NS, e_pad // (NS * CH), CH)
    dst_deg = dst_p.reshape(NC, NS, e_pad // (NC * NS * CH), CH)

    zeros_n1 = jnp.zeros((n1, LW), jnp.float32)
    ones_ch = jnp.ones((CH, LW), jnp.float32)

    pad_rows = ((0, n1 - n), (0, 0))
    xs = jnp.stack([jnp.pad(x1, pad_rows), jnp.pad(x2, pad_rows)])
    ws_l1 = jnp.stack([W1, W2])
    ws_l2 = jnp.stack([W3, W4])
    bs_l1 = jnp.stack([b1, b2])
    bs_l2 = jnp.stack([b3, b4])
    as_l1 = jnp.stack([a1, a3])
    as_l2 = jnp.stack([a2, a4])
    w1p = jnp.pad(w1, pad_rows)
    w2p = jnp.pad(w2, pad_rows)

    degp = _run_deg(dst_deg, zeros_n1, ones_ch, n1)
    tab1, dinv = _run_tc1(xs, ws_l1, degp)
    p1 = _run_prop(tab1.reshape(NC * n1, h), srcoff, dst_prop, n1, h)
    tab2 = _run_tc2(p1, dinv, bs_l1, as_l1, ws_l2)
    p2 = _run_prop(tab2.reshape(NC * n1, h), srcoff, dst_prop, n1, h)
    x = _run_tc3(p2, dinv, bs_l2, as_l2, w1p, w2p)

    return (x[:n], w1, w2)


# speedup vs baseline: 10.3559x; 1.1828x over previous
import jax
import jax.numpy as jnp
from jax import lax
from jax.experimental import pallas as pl
from jax.experimental.pallas import tpu as pltpu
from jax.experimental.pallas import tpu_sc as plsc

NC = 2
NS = 16
CH = 128
GK = 16
LW = 128


def _deg_body(dst_hbm, zeros_hbm, ones_hbm, out_hbm, idx_v, ones_v, acc):
    c = lax.axis_index("c")
    s = lax.axis_index("s")
    n1 = acc.shape[0]
    spt = n1 // NS
    nchunk = idx_v.shape[0]
    pltpu.sync_copy(zeros_hbm.at[pl.ds(s * spt, spt)], acc.at[pl.ds(s * spt, spt)])
    pltpu.sync_copy(ones_hbm, ones_v)
    pltpu.sync_copy(dst_hbm.at[c, s], idx_v)
    plsc.subcore_barrier()

    def step(j, carry):
        pltpu.sync_copy(ones_v, acc.at[idx_v.at[j]], add=True)
        return carry

    lax.fori_loop(0, nchunk, step, 0)
    plsc.subcore_barrier()
    pltpu.sync_copy(acc.at[pl.ds(s * spt, spt)], out_hbm.at[c, pl.ds(s * spt, spt)])


def _prop_body(table_hbm, srcoff_hbm, dst_hbm, out_hbm, idxs_v, idxd_v, rows_v, acc,
               sg0, sg1):
    c = lax.axis_index("c")
    s = lax.axis_index("s")
    n1 = acc.shape[0]
    spt = n1 // NS
    ngroup = srcoff_hbm.shape[2] // GK
    sems = (sg0, sg1)
    pltpu.sync_copy(table_hbm.at[pl.ds(c * n1 + s * spt, spt)],
                    acc.at[pl.ds(s * spt, spt)])
    plsc.subcore_barrier()

    def group(g, carry):
        pltpu.sync_copy(srcoff_hbm.at[c, s, pl.ds(g * GK, GK)], idxs_v)
        pltpu.sync_copy(dst_hbm.at[s, pl.ds(g * GK, GK)], idxd_v)
        descs = [None] * GK
        descs[0] = pltpu.async_copy(table_hbm.at[idxs_v.at[0]], rows_v.at[0], sg0)
        for k in range(GK):
            b = k % 2
            if k + 1 < GK:
                descs[k + 1] = pltpu.async_copy(
                    table_hbm.at[idxs_v.at[k + 1]], rows_v.at[1 - b], sems[1 - b])
            descs[k].wait()
            pltpu.sync_copy(rows_v.at[b], acc.at[idxd_v.at[k]], add=True)
        return carry

    lax.fori_loop(0, ngroup, group, 0)
    plsc.subcore_barrier()
    pltpu.sync_copy(acc.at[pl.ds(s * spt, spt)], out_hbm.at[c, pl.ds(s * spt, spt)])


def _run_deg(dst_deg, zeros_n1, ones_ch, n1):
    mesh = plsc.VectorSubcoreMesh(core_axis_name="c", subcore_axis_name="s")
    dchunk = dst_deg.shape[2]
    return pl.kernel(
        _deg_body,
        out_type=jax.ShapeDtypeStruct((NC, n1, LW), jnp.float32),
        mesh=mesh,
        scratch_types=[
            pltpu.VMEM((dchunk, CH), jnp.int32),
            pltpu.VMEM((CH, LW), jnp.float32),
            pltpu.VMEM_SHARED((n1, LW), jnp.float32),
        ],
    )(dst_deg, zeros_n1, ones_ch)


def _run_prop(table, srcoff, dst_prop, n1, h):
    mesh = plsc.VectorSubcoreMesh(core_axis_name="c", subcore_axis_name="s")
    return pl.kernel(
        _prop_body,
        out_type=jax.ShapeDtypeStruct((NC, n1, h), jnp.float32),
        mesh=mesh,
        scratch_types=[
            pltpu.VMEM((GK, CH), jnp.int32),
            pltpu.VMEM((GK, CH), jnp.int32),
            pltpu.VMEM((2, CH, h), jnp.float32),
            pltpu.VMEM_SHARED((n1, h), jnp.float32),
            pltpu.SemaphoreType.DMA,
            pltpu.SemaphoreType.DMA,
        ],
    )(table, srcoff, dst_prop)


_TCB = 1024


def _tc1_body(x_ref, w_ref, degp_ref, tab_ref, dinv_ref):
    deg = 1.0 + degp_ref[0] + degp_ref[1]
    dinv = lax.rsqrt(deg)[:, 0:1]
    hmat = jnp.dot(x_ref[0], w_ref[0], preferred_element_type=jnp.float32)
    tab_ref[0] = hmat * dinv
    dinv_ref[...] = dinv


def _run_tc1(xs, ws, degp):
    _, n1, d = xs.shape
    h = ws.shape[2]
    nb = n1 // _TCB
    return pl.pallas_call(
        _tc1_body,
        grid=(2, nb),
        in_specs=[
            pl.BlockSpec((1, _TCB, d), lambda c, j: (c, j, 0)),
            pl.BlockSpec((1, d, h), lambda c, j: (c, 0, 0)),
            pl.BlockSpec((2, _TCB, LW), lambda c, j: (0, j, 0)),
        ],
        out_specs=[
            pl.BlockSpec((1, _TCB, h), lambda c, j: (c, j, 0)),
            pl.BlockSpec((_TCB, 1), lambda c, j: (j, 0)),
        ],
        out_shape=[
            jax.ShapeDtypeStruct((2, n1, h), jnp.float32),
            jax.ShapeDtypeStruct((n1, 1), jnp.float32),
        ],
    )(xs, ws, degp)


def _tc2_body(p_ref, dinv_ref, b_ref, a_ref, w_ref, tab_ref):
    c = pl.program_id(0)
    dinv = dinv_ref[...]
    bvec = jnp.where(c == 0, b_ref[0:1], b_ref[1:2])
    avec = jnp.where(c == 0, a_ref[0:1], a_ref[1:2])
    z = p_ref[0] * dinv + bvec
    g = jnp.where(z >= 0, z, avec * z)
    tab_ref[0] = jnp.dot(g, w_ref[0], preferred_element_type=jnp.float32) * dinv


def _run_tc2(p, dinv, bs, as_, ws2):
    _, n1, h = p.shape
    nb = n1 // _TCB
    return pl.pallas_call(
        _tc2_body,
        grid=(2, nb),
        in_specs=[
            pl.BlockSpec((1, _TCB, h), lambda c, j: (c, j, 0)),
            pl.BlockSpec((_TCB, 1), lambda c, j: (j, 0)),
            pl.BlockSpec((2, h), lambda c, j: (0, 0)),
            pl.BlockSpec((2, h), lambda c, j: (0, 0)),
            pl.BlockSpec((1, h, h), lambda c, j: (c, 0, 0)),
        ],
        out_specs=pl.BlockSpec((1, _TCB, h), lambda c, j: (c, j, 0)),
        out_shape=jax.ShapeDtypeStruct((2, n1, h), jnp.float32),
    )(p, dinv, bs, as_, ws2)


def _tc3_body(p_ref, dinv_ref, b_ref, a_ref, w1_ref, w2_ref, x_ref):
    dinv = dinv_ref[...]

    def head(pc, bc, ac):
        z = pc * dinv + bc
        g = jnp.where(z >= 0, z, ac * z)
        nrm = jnp.sqrt(jnp.sum(g * g, axis=1, keepdims=True))
        return g / jnp.maximum(nrm, 1e-12)

    h1 = head(p_ref[0], b_ref[0:1], a_ref[0:1])
    h2 = head(p_ref[1], b_ref[1:2], a_ref[1:2])
    x_ref[...] = h1 * w1_ref[...] + h2 * w2_ref[...]


def _run_tc3(p, dinv, bs2, as2, w1p, w2p):
    _, n1, h = p.shape
    nb = n1 // _TCB
    return pl.pallas_call(
        _tc3_body,
        grid=(nb,),
        in_specs=[
            pl.BlockSpec((2, _TCB, h), lambda j: (0, j, 0)),
            pl.BlockSpec((_TCB, 1), lambda j: (j, 0)),
            pl.BlockSpec((2, h), lambda j: (0, 0)),
            pl.BlockSpec((2, h), lambda j: (0, 0)),
            pl.BlockSpec((_TCB, 1), lambda j: (j, 0)),
            pl.BlockSpec((_TCB, 1), lambda j: (j, 0)),
        ],
        out_specs=pl.BlockSpec((_TCB, h), lambda j: (j, 0)),
        out_shape=jax.ShapeDtypeStruct((n1, h), jnp.float32),
    )(p, dinv, bs2, as2, w1p, w2p)


def kernel(x1, x2, edge_index, adata, w, w1, w2,
           W1, b1, W2, b2, W3, b3, W4, b4, a1, a2, a3, a4):
    n, d = x1.shape
    h = W1.shape[1]
    e = edge_index.shape[1]

    n1 = ((n + LW + _TCB - 1) // _TCB) * _TCB

    grp = NS * CH * GK
    e_pad = ((e + grp - 1) // grp) * grp
    src = edge_index[0].astype(jnp.int32)
    dst = edge_index[1].astype(jnp.int32)
    fill = jnp.full((e_pad - e,), n, dtype=jnp.int32)
    src_p = jnp.concatenate([src, fill])
    dst_p = jnp.concatenate([dst, fill])

    srcoff = jnp.stack([src_p, src_p + n1]).reshape(NC, NS, e_pad // (NS * CH), CH)
    dst_prop = dst_p.reshape(NS, e_pad // (NS * CH), CH)
    dst_deg = dst_p.reshape(NC, NS, e_pad // (NC * NS * CH), CH)

    zeros_n1 = jnp.zeros((n1, LW), jnp.float32)
    ones_ch = jnp.ones((CH, LW), jnp.float32)

    pad_rows = ((0, n1 - n), (0, 0))
    xs = jnp.stack([jnp.pad(x1, pad_rows), jnp.pad(x2, pad_rows)])
    ws_l1 = jnp.stack([W1, W2])
    ws_l2 = jnp.stack([W3, W4])
    bs_l1 = jnp.stack([b1, b2])
    bs_l2 = jnp.stack([b3, b4])
    as_l1 = jnp.stack([a1, a3])
    as_l2 = jnp.stack([a2, a4])
    w1p = jnp.pad(w1, pad_rows)
    w2p = jnp.pad(w2, pad_rows)

    degp = _run_deg(dst_deg, zeros_n1, ones_ch, n1)
    tab1, dinv = _run_tc1(xs, ws_l1, degp)
    p1 = _run_prop(tab1.reshape(NC * n1, h), srcoff, dst_prop, n1, h)
    tab2 = _run_tc2(p1, dinv, bs_l1, as_l1, ws_l2)
    p2 = _run_prop(tab2.reshape(NC * n1, h), srcoff, dst_prop, n1, h)
    x = _run_tc3(p2, dinv, bs_l2, as_l2, w1p, w2p)

    return (x[:n], w1, w2)


# speedup vs baseline: 10.5436x; 1.0181x over previous
import jax
import jax.numpy as jnp
from jax import lax
from jax.experimental import pallas as pl
from jax.experimental.pallas import tpu as pltpu
from jax.experimental.pallas import tpu_sc as plsc

NC = 2
NS = 16
CH = 128
GK = 32
LW = 128


def _deg_body(dst_hbm, zeros_hbm, ones_hbm, out_hbm, idx_v, ones_v, acc):
    c = lax.axis_index("c")
    s = lax.axis_index("s")
    n1 = acc.shape[0]
    spt = n1 // NS
    nchunk = idx_v.shape[0]
    pltpu.sync_copy(zeros_hbm.at[pl.ds(s * spt, spt)], acc.at[pl.ds(s * spt, spt)])
    pltpu.sync_copy(ones_hbm, ones_v)
    pltpu.sync_copy(dst_hbm.at[c, s], idx_v)
    plsc.subcore_barrier()

    def step(j, carry):
        pltpu.sync_copy(ones_v, acc.at[idx_v.at[j]], add=True)
        return carry

    lax.fori_loop(0, nchunk, step, 0)
    plsc.subcore_barrier()
    pltpu.sync_copy(acc.at[pl.ds(s * spt, spt)], out_hbm.at[c, pl.ds(s * spt, spt)])


def _prop_body(table_hbm, srcoff_hbm, dst_hbm, out_hbm, idxs_v, idxd_v, rows_v, acc,
               sg0, sg1):
    c = lax.axis_index("c")
    s = lax.axis_index("s")
    n1 = acc.shape[0]
    spt = n1 // NS
    ngroup = srcoff_hbm.shape[2] // GK
    sems = (sg0, sg1)
    pltpu.sync_copy(table_hbm.at[pl.ds(c * n1 + s * spt, spt)],
                    acc.at[pl.ds(s * spt, spt)])
    plsc.subcore_barrier()

    def group(g, carry):
        pltpu.sync_copy(srcoff_hbm.at[c, s, pl.ds(g * GK, GK)], idxs_v)
        pltpu.sync_copy(dst_hbm.at[s, pl.ds(g * GK, GK)], idxd_v)
        descs = [None] * GK
        descs[0] = pltpu.async_copy(table_hbm.at[idxs_v.at[0]], rows_v.at[0], sg0)
        for k in range(GK):
            b = k % 2
            if k + 1 < GK:
                descs[k + 1] = pltpu.async_copy(
                    table_hbm.at[idxs_v.at[k + 1]], rows_v.at[1 - b], sems[1 - b])
            descs[k].wait()
            pltpu.sync_copy(rows_v.at[b], acc.at[idxd_v.at[k]], add=True)
        return carry

    lax.fori_loop(0, ngroup, group, 0)
    plsc.subcore_barrier()
    pltpu.sync_copy(acc.at[pl.ds(s * spt, spt)], out_hbm.at[c, pl.ds(s * spt, spt)])


def _run_deg(dst_deg, zeros_n1, ones_ch, n1):
    mesh = plsc.VectorSubcoreMesh(core_axis_name="c", subcore_axis_name="s")
    dchunk = dst_deg.shape[2]
    return pl.kernel(
        _deg_body,
        out_type=jax.ShapeDtypeStruct((NC, n1, LW), jnp.float32),
        mesh=mesh,
        scratch_types=[
            pltpu.VMEM((dchunk, CH), jnp.int32),
            pltpu.VMEM((CH, LW), jnp.float32),
            pltpu.VMEM_SHARED((n1, LW), jnp.float32),
        ],
    )(dst_deg, zeros_n1, ones_ch)


def _run_prop(table, srcoff, dst_prop, n1, h):
    mesh = plsc.VectorSubcoreMesh(core_axis_name="c", subcore_axis_name="s")
    return pl.kernel(
        _prop_body,
        out_type=jax.ShapeDtypeStruct((NC, n1, h), jnp.float32),
        mesh=mesh,
        scratch_types=[
            pltpu.VMEM((GK, CH), jnp.int32),
            pltpu.VMEM((GK, CH), jnp.int32),
            pltpu.VMEM((2, CH, h), jnp.float32),
            pltpu.VMEM_SHARED((n1, h), jnp.float32),
            pltpu.SemaphoreType.DMA,
            pltpu.SemaphoreType.DMA,
        ],
    )(table, srcoff, dst_prop)


_TCB = 1024


def _tc1_body(x_ref, w_ref, degp_ref, tab_ref, dinv_ref):
    deg = 1.0 + degp_ref[0] + degp_ref[1]
    dinv = lax.rsqrt(deg)[:, 0:1]
    hmat = jnp.dot(x_ref[0], w_ref[0], preferred_element_type=jnp.float32)
    tab_ref[0] = hmat * dinv
    dinv_ref[...] = dinv


def _run_tc1(xs, ws, degp):
    _, n1, d = xs.shape
    h = ws.shape[2]
    nb = n1 // _TCB
    return pl.pallas_call(
        _tc1_body,
        grid=(2, nb),
        in_specs=[
            pl.BlockSpec((1, _TCB, d), lambda c, j: (c, j, 0)),
            pl.BlockSpec((1, d, h), lambda c, j: (c, 0, 0)),
            pl.BlockSpec((2, _TCB, LW), lambda c, j: (0, j, 0)),
        ],
        out_specs=[
            pl.BlockSpec((1, _TCB, h), lambda c, j: (c, j, 0)),
            pl.BlockSpec((_TCB, 1), lambda c, j: (j, 0)),
        ],
        out_shape=[
            jax.ShapeDtypeStruct((2, n1, h), jnp.float32),
            jax.ShapeDtypeStruct((n1, 1), jnp.float32),
        ],
    )(xs, ws, degp)


def _tc2_body(p_ref, dinv_ref, b_ref, a_ref, w_ref, tab_ref):
    c = pl.program_id(0)
    dinv = dinv_ref[...]
    bvec = jnp.where(c == 0, b_ref[0:1], b_ref[1:2])
    avec = jnp.where(c == 0, a_ref[0:1], a_ref[1:2])
    z = p_ref[0] * dinv + bvec
    g = jnp.where(z >= 0, z, avec * z)
    tab_ref[0] = jnp.dot(g, w_ref[0], preferred_element_type=jnp.float32) * dinv


def _run_tc2(p, dinv, bs, as_, ws2):
    _, n1, h = p.shape
    nb = n1 // _TCB
    return pl.pallas_call(
        _tc2_body,
        grid=(2, nb),
        in_specs=[
            pl.BlockSpec((1, _TCB, h), lambda c, j: (c, j, 0)),
            pl.BlockSpec((_TCB, 1), lambda c, j: (j, 0)),
            pl.BlockSpec((2, h), lambda c, j: (0, 0)),
            pl.BlockSpec((2, h), lambda c, j: (0, 0)),
            pl.BlockSpec((1, h, h), lambda c, j: (c, 0, 0)),
        ],
        out_specs=pl.BlockSpec((1, _TCB, h), lambda c, j: (c, j, 0)),
        out_shape=jax.ShapeDtypeStruct((2, n1, h), jnp.float32),
    )(p, dinv, bs, as_, ws2)


def _tc3_body(p_ref, dinv_ref, b_ref, a_ref, w1_ref, w2_ref, x_ref):
    dinv = dinv_ref[...]

    def head(pc, bc, ac):
        z = pc * dinv + bc
        g = jnp.where(z >= 0, z, ac * z)
        nrm = jnp.sqrt(jnp.sum(g * g, axis=1, keepdims=True))
        return g / jnp.maximum(nrm, 1e-12)

    h1 = head(p_ref[0], b_ref[0:1], a_ref[0:1])
    h2 = head(p_ref[1], b_ref[1:2], a_ref[1:2])
    x_ref[...] = h1 * w1_ref[...] + h2 * w2_ref[...]


def _run_tc3(p, dinv, bs2, as2, w1p, w2p):
    _, n1, h = p.shape
    nb = n1 // _TCB
    return pl.pallas_call(
        _tc3_body,
        grid=(nb,),
        in_specs=[
            pl.BlockSpec((2, _TCB, h), lambda j: (0, j, 0)),
            pl.BlockSpec((_TCB, 1), lambda j: (j, 0)),
            pl.BlockSpec((2, h), lambda j: (0, 0)),
            pl.BlockSpec((2, h), lambda j: (0, 0)),
            pl.BlockSpec((_TCB, 1), lambda j: (j, 0)),
            pl.BlockSpec((_TCB, 1), lambda j: (j, 0)),
        ],
        out_specs=pl.BlockSpec((_TCB, h), lambda j: (j, 0)),
        out_shape=jax.ShapeDtypeStruct((n1, h), jnp.float32),
    )(p, dinv, bs2, as2, w1p, w2p)


def kernel(x1, x2, edge_index, adata, w, w1, w2,
           W1, b1, W2, b2, W3, b3, W4, b4, a1, a2, a3, a4):
    n, d = x1.shape
    h = W1.shape[1]
    e = edge_index.shape[1]

    n1 = ((n + LW + _TCB - 1) // _TCB) * _TCB

    grp = NS * CH * GK
    e_pad = ((e + grp - 1) // grp) * grp
    src = edge_index[0].astype(jnp.int32)
    dst = edge_index[1].astype(jnp.int32)
    fill = jnp.full((e_pad - e,), n, dtype=jnp.int32)
    src_p = jnp.concatenate([src, fill])
    dst_p = jnp.concatenate([dst, fill])

    srcoff = jnp.stack([src_p, src_p + n1]).reshape(NC, NS, e_pad // (NS * CH), CH)
    dst_prop = dst_p.reshape(NS, e_pad // (NS * CH), CH)
    dst_deg = dst_p.reshape(NC, NS, e_pad // (NC * NS * CH), CH)

    zeros_n1 = jnp.zeros((n1, LW), jnp.float32)
    ones_ch = jnp.ones((CH, LW), jnp.float32)

    pad_rows = ((0, n1 - n), (0, 0))
    xs = jnp.stack([jnp.pad(x1, pad_rows), jnp.pad(x2, pad_rows)])
    ws_l1 = jnp.stack([W1, W2])
    ws_l2 = jnp.stack([W3, W4])
    bs_l1 = jnp.stack([b1, b2])
    bs_l2 = jnp.stack([b3, b4])
    as_l1 = jnp.stack([a1, a3])
    as_l2 = jnp.stack([a2, a4])
    w1p = jnp.pad(w1, pad_rows)
    w2p = jnp.pad(w2, pad_rows)

    degp = _run_deg(dst_deg, zeros_n1, ones_ch, n1)
    tab1, dinv = _run_tc1(xs, ws_l1, degp)
    p1 = _run_prop(tab1.reshape(NC * n1, h), srcoff, dst_prop, n1, h)
    tab2 = _run_tc2(p1, dinv, bs_l1, as_l1, ws_l2)
    p2 = _run_prop(tab2.reshape(NC * n1, h), srcoff, dst_prop, n1, h)
    x = _run_tc3(p2, dinv, bs_l2, as_l2, w1p, w2p)

    return (x[:n], w1, w2)


# speedup vs baseline: 10.5761x; 1.0031x over previous
import jax
import jax.numpy as jnp
from jax import lax
from jax.experimental import pallas as pl
from jax.experimental.pallas import tpu as pltpu
from jax.experimental.pallas import tpu_sc as plsc

NC = 2
NS = 16
CH = 128
GK = 40
LW = 128


def _deg_body(dst_hbm, zeros_hbm, ones_hbm, out_hbm, idx_v, ones_v, acc):
    c = lax.axis_index("c")
    s = lax.axis_index("s")
    n1 = acc.shape[0]
    spt = n1 // NS
    nchunk = idx_v.shape[0]
    pltpu.sync_copy(zeros_hbm.at[pl.ds(s * spt, spt)], acc.at[pl.ds(s * spt, spt)])
    pltpu.sync_copy(ones_hbm, ones_v)
    pltpu.sync_copy(dst_hbm.at[c, s], idx_v)
    plsc.subcore_barrier()

    def step(j, carry):
        pltpu.sync_copy(ones_v, acc.at[idx_v.at[j]], add=True)
        return carry

    lax.fori_loop(0, nchunk, step, 0)
    plsc.subcore_barrier()
    pltpu.sync_copy(acc.at[pl.ds(s * spt, spt)], out_hbm.at[c, pl.ds(s * spt, spt)])


def _prop_body(table_hbm, srcoff_hbm, dst_hbm, out_hbm, idxs_v, idxd_v, rows_v, acc,
               sg0, sg1):
    c = lax.axis_index("c")
    s = lax.axis_index("s")
    n1 = acc.shape[0]
    spt = n1 // NS
    ngroup = srcoff_hbm.shape[2] // GK
    sems = (sg0, sg1)
    pltpu.sync_copy(table_hbm.at[pl.ds(c * n1 + s * spt, spt)],
                    acc.at[pl.ds(s * spt, spt)])
    plsc.subcore_barrier()

    def group(g, carry):
        pltpu.sync_copy(srcoff_hbm.at[c, s, pl.ds(g * GK, GK)], idxs_v)
        pltpu.sync_copy(dst_hbm.at[s, pl.ds(g * GK, GK)], idxd_v)
        descs = [None] * GK
        descs[0] = pltpu.async_copy(table_hbm.at[idxs_v.at[0]], rows_v.at[0], sg0)
        for k in range(GK):
            b = k % 2
            if k + 1 < GK:
                descs[k + 1] = pltpu.async_copy(
                    table_hbm.at[idxs_v.at[k + 1]], rows_v.at[1 - b], sems[1 - b])
            descs[k].wait()
            pltpu.sync_copy(rows_v.at[b], acc.at[idxd_v.at[k]], add=True)
        return carry

    lax.fori_loop(0, ngroup, group, 0)
    plsc.subcore_barrier()
    pltpu.sync_copy(acc.at[pl.ds(s * spt, spt)], out_hbm.at[c, pl.ds(s * spt, spt)])


def _run_deg(dst_deg, zeros_n1, ones_ch, n1):
    mesh = plsc.VectorSubcoreMesh(core_axis_name="c", subcore_axis_name="s")
    dchunk = dst_deg.shape[2]
    return pl.kernel(
        _deg_body,
        out_type=jax.ShapeDtypeStruct((NC, n1, LW), jnp.float32),
        mesh=mesh,
        scratch_types=[
            pltpu.VMEM((dchunk, CH), jnp.int32),
            pltpu.VMEM((CH, LW), jnp.float32),
            pltpu.VMEM_SHARED((n1, LW), jnp.float32),
        ],
    )(dst_deg, zeros_n1, ones_ch)


def _run_prop(table, srcoff, dst_prop, n1, h):
    mesh = plsc.VectorSubcoreMesh(core_axis_name="c", subcore_axis_name="s")
    return pl.kernel(
        _prop_body,
        out_type=jax.ShapeDtypeStruct((NC, n1, h), jnp.float32),
        mesh=mesh,
        scratch_types=[
            pltpu.VMEM((GK, CH), jnp.int32),
            pltpu.VMEM((GK, CH), jnp.int32),
            pltpu.VMEM((2, CH, h), jnp.float32),
            pltpu.VMEM_SHARED((n1, h), jnp.float32),
            pltpu.SemaphoreType.DMA,
            pltpu.SemaphoreType.DMA,
        ],
    )(table, srcoff, dst_prop)


_TCB = 1024


def _tc1_body(x_ref, w_ref, degp_ref, tab_ref, dinv_ref):
    deg = 1.0 + degp_ref[0] + degp_ref[1]
    dinv = lax.rsqrt(deg)[:, 0:1]
    hmat = jnp.dot(x_ref[0], w_ref[0], preferred_element_type=jnp.float32)
    tab_ref[0] = hmat * dinv
    dinv_ref[...] = dinv


def _run_tc1(xs, ws, degp):
    _, n1, d = xs.shape
    h = ws.shape[2]
    nb = n1 // _TCB
    return pl.pallas_call(
        _tc1_body,
        grid=(2, nb),
        in_specs=[
            pl.BlockSpec((1, _TCB, d), lambda c, j: (c, j, 0)),
            pl.BlockSpec((1, d, h), lambda c, j: (c, 0, 0)),
            pl.BlockSpec((2, _TCB, LW), lambda c, j: (0, j, 0)),
        ],
        out_specs=[
            pl.BlockSpec((1, _TCB, h), lambda c, j: (c, j, 0)),
            pl.BlockSpec((_TCB, 1), lambda c, j: (j, 0)),
        ],
        out_shape=[
            jax.ShapeDtypeStruct((2, n1, h), jnp.float32),
            jax.ShapeDtypeStruct((n1, 1), jnp.float32),
        ],
    )(xs, ws, degp)


def _tc2_body(p_ref, dinv_ref, b_ref, a_ref, w_ref, tab_ref):
    c = pl.program_id(0)
    dinv = dinv_ref[...]
    bvec = jnp.where(c == 0, b_ref[0:1], b_ref[1:2])
    avec = jnp.where(c == 0, a_ref[0:1], a_ref[1:2])
    z = p_ref[0] * dinv + bvec
    g = jnp.where(z >= 0, z, avec * z)
    tab_ref[0] = jnp.dot(g, w_ref[0], preferred_element_type=jnp.float32) * dinv


def _run_tc2(p, dinv, bs, as_, ws2):
    _, n1, h = p.shape
    nb = n1 // _TCB
    return pl.pallas_call(
        _tc2_body,
        grid=(2, nb),
        in_specs=[
            pl.BlockSpec((1, _TCB, h), lambda c, j: (c, j, 0)),
            pl.BlockSpec((_TCB, 1), lambda c, j: (j, 0)),
            pl.BlockSpec((2, h), lambda c, j: (0, 0)),
            pl.BlockSpec((2, h), lambda c, j: (0, 0)),
            pl.BlockSpec((1, h, h), lambda c, j: (c, 0, 0)),
        ],
        out_specs=pl.BlockSpec((1, _TCB, h), lambda c, j: (c, j, 0)),
        out_shape=jax.ShapeDtypeStruct((2, n1, h), jnp.float32),
    )(p, dinv, bs, as_, ws2)


def _tc3_body(p_ref, dinv_ref, b_ref, a_ref, w1_ref, w2_ref, x_ref):
    dinv = dinv_ref[...]

    def head(pc, bc, ac):
        z = pc * dinv + bc
        g = jnp.where(z >= 0, z, ac * z)
        nrm = jnp.sqrt(jnp.sum(g * g, axis=1, keepdims=True))
        return g / jnp.maximum(nrm, 1e-12)

    h1 = head(p_ref[0], b_ref[0:1], a_ref[0:1])
    h2 = head(p_ref[1], b_ref[1:2], a_ref[1:2])
    x_ref[...] = h1 * w1_ref[...] + h2 * w2_ref[...]


def _run_tc3(p, dinv, bs2, as2, w1p, w2p):
    _, n1, h = p.shape
    nb = n1 // _TCB
    return pl.pallas_call(
        _tc3_body,
        grid=(nb,),
        in_specs=[
            pl.BlockSpec((2, _TCB, h), lambda j: (0, j, 0)),
            pl.BlockSpec((_TCB, 1), lambda j: (j, 0)),
            pl.BlockSpec((2, h), lambda j: (0, 0)),
            pl.BlockSpec((2, h), lambda j: (0, 0)),
            pl.BlockSpec((_TCB, 1), lambda j: (j, 0)),
            pl.BlockSpec((_TCB, 1), lambda j: (j, 0)),
        ],
        out_specs=pl.BlockSpec((_TCB, h), lambda j: (j, 0)),
        out_shape=jax.ShapeDtypeStruct((n1, h), jnp.float32),
    )(p, dinv, bs2, as2, w1p, w2p)


def kernel(x1, x2, edge_index, adata, w, w1, w2,
           W1, b1, W2, b2, W3, b3, W4, b4, a1, a2, a3, a4):
    n, d = x1.shape
    h = W1.shape[1]
    e = edge_index.shape[1]

    n1 = ((n + LW + _TCB - 1) // _TCB) * _TCB

    grp = NS * CH * GK
    e_pad = ((e + grp - 1) // grp) * grp
    src = edge_index[0].astype(jnp.int32)
    dst = edge_index[1].astype(jnp.int32)
    fill = jnp.full((e_pad - e,), n, dtype=jnp.int32)
    src_p = jnp.concatenate([src, fill])
    dst_p = jnp.concatenate([dst, fill])

    srcoff = jnp.stack([src_p, src_p + n1]).reshape(NC, NS, e_pad // (NS * CH), CH)
    dst_prop = dst_p.reshape(NS, e_pad // (NS * CH), CH)
    dst_deg = dst_p.reshape(NC, NS, e_pad // (NC * NS * CH), CH)

    zeros_n1 = jnp.zeros((n1, LW), jnp.float32)
    ones_ch = jnp.ones((CH, LW), jnp.float32)

    pad_rows = ((0, n1 - n), (0, 0))
    xs = jnp.stack([jnp.pad(x1, pad_rows), jnp.pad(x2, pad_rows)])
    ws_l1 = jnp.stack([W1, W2])
    ws_l2 = jnp.stack([W3, W4])
    bs_l1 = jnp.stack([b1, b2])
    bs_l2 = jnp.stack([b3, b4])
    as_l1 = jnp.stack([a1, a3])
    as_l2 = jnp.stack([a2, a4])
    w1p = jnp.pad(w1, pad_rows)
    w2p = jnp.pad(w2, pad_rows)

    degp = _run_deg(dst_deg, zeros_n1, ones_ch, n1)
    tab1, dinv = _run_tc1(xs, ws_l1, degp)
    p1 = _run_prop(tab1.reshape(NC * n1, h), srcoff, dst_prop, n1, h)
    tab2 = _run_tc2(p1, dinv, bs_l1, as_l1, ws_l2)
    p2 = _run_prop(tab2.reshape(NC * n1, h), srcoff, dst_prop, n1, h)
    x = _run_tc3(p2, dinv, bs_l2, as_l2, w1p, w2p)

    return (x[:n], w1, w2)


# speedup vs baseline: 10.7042x; 1.0121x over previous
import jax
import jax.numpy as jnp
from jax import lax
from jax.experimental import pallas as pl
from jax.experimental.pallas import tpu as pltpu
from jax.experimental.pallas import tpu_sc as plsc

NC = 2
NS = 16
CH = 128
GK = 16
LW = 128


def _deg_body(dst_hbm, zeros_hbm, ones_hbm, out_hbm, idx_v, ones_v, acc):
    c = lax.axis_index("c")
    s = lax.axis_index("s")
    n1 = acc.shape[0]
    spt = n1 // NS
    nchunk = idx_v.shape[0]
    pltpu.sync_copy(zeros_hbm.at[pl.ds(s * spt, spt)], acc.at[pl.ds(s * spt, spt)])
    pltpu.sync_copy(ones_hbm, ones_v)
    pltpu.sync_copy(dst_hbm.at[c, s], idx_v)
    plsc.subcore_barrier()

    def step(j, carry):
        pltpu.sync_copy(ones_v, acc.at[idx_v.at[j]], add=True)
        return carry

    lax.fori_loop(0, nchunk, step, 0)
    plsc.subcore_barrier()
    pltpu.sync_copy(acc.at[pl.ds(s * spt, spt)], out_hbm.at[c, pl.ds(s * spt, spt)])


def _prop_body(table_hbm, srcoff_hbm, dst_hbm, out_hbm, idxs_v, idxd_v, rows_v, acc,
               sg0, sg1, si0, si1):
    c = lax.axis_index("c")
    s = lax.axis_index("s")
    n1 = acc.shape[0]
    spt = n1 // NS
    nch = srcoff_hbm.shape[2]
    ngroup = nch // GK
    sems = (sg0, sg1)
    pltpu.sync_copy(table_hbm.at[pl.ds(c * n1 + s * spt, spt)],
                    acc.at[pl.ds(s * spt, spt)])
    plsc.subcore_barrier()

    def stage(g):
        slot = g % 2
        return (pltpu.async_copy(srcoff_hbm.at[c, s, pl.ds(g * GK, GK)],
                                 idxs_v.at[slot], si0),
                pltpu.async_copy(dst_hbm.at[s, pl.ds(g * GK, GK)],
                                 idxd_v.at[slot], si1))

    idesc = [None] * ngroup
    for d in stage(0):
        d.wait()
    if ngroup > 1:
        idesc[1] = stage(1)
    waited = [True] + [False] * (ngroup - 1)
    descs = [None] * nch
    descs[0] = pltpu.async_copy(table_hbm.at[idxs_v.at[0, 0]], rows_v.at[0], sg0)
    for k in range(nch):
        g, kk = divmod(k, GK)
        b = k % 2
        if k + 1 < nch:
            g2, kk2 = divmod(k + 1, GK)
            if not waited[g2]:
                for d in idesc[g2]:
                    d.wait()
                waited[g2] = True
            descs[k + 1] = pltpu.async_copy(
                table_hbm.at[idxs_v.at[g2 % 2, kk2]], rows_v.at[1 - b], sems[1 - b])
        descs[k].wait()
        pltpu.sync_copy(rows_v.at[b], acc.at[idxd_v.at[g % 2, kk]], add=True)
        if kk == GK - 1 and g + 2 < ngroup:
            idesc[g + 2] = stage(g + 2)

    plsc.subcore_barrier()
    pltpu.sync_copy(acc.at[pl.ds(s * spt, spt)], out_hbm.at[c, pl.ds(s * spt, spt)])


def _run_deg(dst_deg, zeros_n1, ones_ch, n1):
    mesh = plsc.VectorSubcoreMesh(core_axis_name="c", subcore_axis_name="s")
    dchunk = dst_deg.shape[2]
    return pl.kernel(
        _deg_body,
        out_type=jax.ShapeDtypeStruct((NC, n1, LW), jnp.float32),
        mesh=mesh,
        scratch_types=[
            pltpu.VMEM((dchunk, CH), jnp.int32),
            pltpu.VMEM((CH, LW), jnp.float32),
            pltpu.VMEM_SHARED((n1, LW), jnp.float32),
        ],
    )(dst_deg, zeros_n1, ones_ch)


def _run_prop(table, srcoff, dst_prop, n1, h):
    mesh = plsc.VectorSubcoreMesh(core_axis_name="c", subcore_axis_name="s")
    return pl.kernel(
        _prop_body,
        out_type=jax.ShapeDtypeStruct((NC, n1, h), jnp.float32),
        mesh=mesh,
        scratch_types=[
            pltpu.VMEM((2, GK, CH), jnp.int32),
            pltpu.VMEM((2, GK, CH), jnp.int32),
            pltpu.VMEM((2, CH, h), jnp.float32),
            pltpu.VMEM_SHARED((n1, h), jnp.float32),
            pltpu.SemaphoreType.DMA,
            pltpu.SemaphoreType.DMA,
            pltpu.SemaphoreType.DMA,
            pltpu.SemaphoreType.DMA,
        ],
    )(table, srcoff, dst_prop)


_TCB = 1024


def _tc1_body(x_ref, w_ref, degp_ref, tab_ref, dinv_ref):
    deg = 1.0 + degp_ref[0] + degp_ref[1]
    dinv = lax.rsqrt(deg)[:, 0:1]
    hmat = jnp.dot(x_ref[0], w_ref[0], preferred_element_type=jnp.float32)
    tab_ref[0] = hmat * dinv
    dinv_ref[...] = dinv


def _run_tc1(xs, ws, degp):
    _, n1, d = xs.shape
    h = ws.shape[2]
    nb = n1 // _TCB
    return pl.pallas_call(
        _tc1_body,
        grid=(2, nb),
        in_specs=[
            pl.BlockSpec((1, _TCB, d), lambda c, j: (c, j, 0)),
            pl.BlockSpec((1, d, h), lambda c, j: (c, 0, 0)),
            pl.BlockSpec((2, _TCB, LW), lambda c, j: (0, j, 0)),
        ],
        out_specs=[
            pl.BlockSpec((1, _TCB, h), lambda c, j: (c, j, 0)),
            pl.BlockSpec((_TCB, 1), lambda c, j: (j, 0)),
        ],
        out_shape=[
            jax.ShapeDtypeStruct((2, n1, h), jnp.float32),
            jax.ShapeDtypeStruct((n1, 1), jnp.float32),
        ],
    )(xs, ws, degp)


def _tc2_body(p_ref, dinv_ref, b_ref, a_ref, w_ref, tab_ref):
    c = pl.program_id(0)
    dinv = dinv_ref[...]
    bvec = jnp.where(c == 0, b_ref[0:1], b_ref[1:2])
    avec = jnp.where(c == 0, a_ref[0:1], a_ref[1:2])
    z = p_ref[0] * dinv + bvec
    g = jnp.where(z >= 0, z, avec * z)
    tab_ref[0] = jnp.dot(g, w_ref[0], preferred_element_type=jnp.float32) * dinv


def _run_tc2(p, dinv, bs, as_, ws2):
    _, n1, h = p.shape
    nb = n1 // _TCB
    return pl.pallas_call(
        _tc2_body,
        grid=(2, nb),
        in_specs=[
            pl.BlockSpec((1, _TCB, h), lambda c, j: (c, j, 0)),
            pl.BlockSpec((_TCB, 1), lambda c, j: (j, 0)),
            pl.BlockSpec((2, h), lambda c, j: (0, 0)),
            pl.BlockSpec((2, h), lambda c, j: (0, 0)),
            pl.BlockSpec((1, h, h), lambda c, j: (c, 0, 0)),
        ],
        out_specs=pl.BlockSpec((1, _TCB, h), lambda c, j: (c, j, 0)),
        out_shape=jax.ShapeDtypeStruct((2, n1, h), jnp.float32),
    )(p, dinv, bs, as_, ws2)


def _tc3_body(p_ref, dinv_ref, b_ref, a_ref, w1_ref, w2_ref, x_ref):
    dinv = dinv_ref[...]

    def head(pc, bc, ac):
        z = pc * dinv + bc
        g = jnp.where(z >= 0, z, ac * z)
        nrm = jnp.sqrt(jnp.sum(g * g, axis=1, keepdims=True))
        return g / jnp.maximum(nrm, 1e-12)

    h1 = head(p_ref[0], b_ref[0:1], a_ref[0:1])
    h2 = head(p_ref[1], b_ref[1:2], a_ref[1:2])
    x_ref[...] = h1 * w1_ref[...] + h2 * w2_ref[...]


def _run_tc3(p, dinv, bs2, as2, w1p, w2p):
    _, n1, h = p.shape
    nb = n1 // _TCB
    return pl.pallas_call(
        _tc3_body,
        grid=(nb,),
        in_specs=[
            pl.BlockSpec((2, _TCB, h), lambda j: (0, j, 0)),
            pl.BlockSpec((_TCB, 1), lambda j: (j, 0)),
            pl.BlockSpec((2, h), lambda j: (0, 0)),
            pl.BlockSpec((2, h), lambda j: (0, 0)),
            pl.BlockSpec((_TCB, 1), lambda j: (j, 0)),
            pl.BlockSpec((_TCB, 1), lambda j: (j, 0)),
        ],
        out_specs=pl.BlockSpec((_TCB, h), lambda j: (j, 0)),
        out_shape=jax.ShapeDtypeStruct((n1, h), jnp.float32),
    )(p, dinv, bs2, as2, w1p, w2p)


def kernel(x1, x2, edge_index, adata, w, w1, w2,
           W1, b1, W2, b2, W3, b3, W4, b4, a1, a2, a3, a4):
    n, d = x1.shape
    h = W1.shape[1]
    e = edge_index.shape[1]

    n1 = ((n + LW + _TCB - 1) // _TCB) * _TCB

    grp = NS * CH * GK
    e_pad = ((e + grp - 1) // grp) * grp
    src = edge_index[0].astype(jnp.int32)
    dst = edge_index[1].astype(jnp.int32)
    fill = jnp.full((e_pad - e,), n, dtype=jnp.int32)
    src_p = jnp.concatenate([src, fill])
    dst_p = jnp.concatenate([dst, fill])

    srcoff = jnp.stack([src_p, src_p + n1]).reshape(NC, NS, e_pad // (NS * CH), CH)
    dst_prop = dst_p.reshape(NS, e_pad // (NS * CH), CH)
    dst_deg = dst_p.reshape(NC, NS, e_pad // (NC * NS * CH), CH)

    zeros_n1 = jnp.zeros((n1, LW), jnp.float32)
    ones_ch = jnp.ones((CH, LW), jnp.float32)

    pad_rows = ((0, n1 - n), (0, 0))
    xs = jnp.stack([jnp.pad(x1, pad_rows), jnp.pad(x2, pad_rows)])
    ws_l1 = jnp.stack([W1, W2])
    ws_l2 = jnp.stack([W3, W4])
    bs_l1 = jnp.stack([b1, b2])
    bs_l2 = jnp.stack([b3, b4])
    as_l1 = jnp.stack([a1, a3])
    as_l2 = jnp.stack([a2, a4])
    w1p = jnp.pad(w1, pad_rows)
    w2p = jnp.pad(w2, pad_rows)

    degp = _run_deg(dst_deg, zeros_n1, ones_ch, n1)
    tab1, dinv = _run_tc1(xs, ws_l1, degp)
    p1 = _run_prop(tab1.reshape(NC * n1, h), srcoff, dst_prop, n1, h)
    tab2 = _run_tc2(p1, dinv, bs_l1, as_l1, ws_l2)
    p2 = _run_prop(tab2.reshape(NC * n1, h), srcoff, dst_prop, n1, h)
    x = _run_tc3(p2, dinv, bs_l2, as_l2, w1p, w2p)

    return (x[:n], w1, w2)
